# Initial kernel scaffold; baseline (speedup 1.0000x reference)
import jax
import jax.numpy as jnp
from jax.experimental import pallas as pl


def kernel(edge_feat, node_feat, edge_index):
    raise NotImplementedError("write your pallas kernel here")


# speedup vs baseline: 4.5266x; 4.5266x over previous
import functools

import jax
import jax.numpy as jnp
from jax import lax
from jax.experimental import pallas as pl
from jax.experimental.pallas import tpu as pltpu
from jax.experimental.pallas import tpu_sc as plsc


def _make_sc_kernel(E, N, D, NW, CHUNK):
    e_per_w = E // NW
    n_chunks = e_per_w // CHUNK
    mesh = plsc.VectorSubcoreMesh(core_axis_name="c", subcore_axis_name="s")

    @functools.partial(
        pl.kernel,
        mesh=mesh,
        out_type=jax.ShapeDtypeStruct((E, 3 * D), jnp.float32),
        scratch_types=[
            pltpu.VMEM((e_per_w,), jnp.int32),
            pltpu.VMEM((e_per_w,), jnp.int32),
            pltpu.VMEM((CHUNK, D), jnp.float32),
            pltpu.VMEM((CHUNK, D), jnp.float32),
            pltpu.VMEM((CHUNK, D), jnp.float32),
            pltpu.SemaphoreType.DMA,
            pltpu.SemaphoreType.DMA,
        ],
    )
    def sc_gather(edge_hbm, node_hbm, src_hbm, dst_hbm, out_hbm,
                  srcv, dstv, ev, sv, dv, sem_s, sem_d):
        wid = lax.axis_index("s") * 2 + lax.axis_index("c")
        base = wid * e_per_w
        pltpu.sync_copy(src_hbm.at[pl.ds(base, e_per_w)], srcv)
        pltpu.sync_copy(dst_hbm.at[pl.ds(base, e_per_w)], dstv)

        def body(i, carry):
            off = i * CHUNK
            g1 = pltpu.async_copy(
                node_hbm.at[srcv.at[pl.ds(off, CHUNK)]], sv, sem_s)
            g2 = pltpu.async_copy(
                node_hbm.at[dstv.at[pl.ds(off, CHUNK)]], dv, sem_d)
            row = base + off
            pltpu.sync_copy(edge_hbm.at[pl.ds(row, CHUNK)], ev)
            pltpu.sync_copy(ev, out_hbm.at[pl.ds(row, CHUNK), pl.ds(0, D)])
            g1.wait()
            pltpu.sync_copy(sv, out_hbm.at[pl.ds(row, CHUNK), pl.ds(D, D)])
            g2.wait()
            pltpu.sync_copy(dv, out_hbm.at[pl.ds(row, CHUNK), pl.ds(2 * D, D)])
            return carry

        lax.fori_loop(0, n_chunks, body, 0)

    return sc_gather


def kernel(edge_feat, node_feat, edge_index):
    E, D = edge_feat.shape
    N = node_feat.shape[0]
    src = edge_index[0].astype(jnp.int32)
    dst = edge_index[1].astype(jnp.int32)
    fn = _make_sc_kernel(E, N, D, NW=32, CHUNK=80)
    return fn(edge_feat, node_feat, src, dst)


# speedup vs baseline: 5.1127x; 1.1295x over previous
import functools

import jax
import jax.numpy as jnp
from jax import lax
from jax.experimental import pallas as pl
from jax.experimental.pallas import tpu as pltpu
from jax.experimental.pallas import tpu_sc as plsc


def _make_sc_kernel(E, N, D, NW, CHUNK):
    e_per_w = E // NW
    n_chunks = pl.cdiv(e_per_w, CHUNK)
    mesh = plsc.VectorSubcoreMesh(core_axis_name="c", subcore_axis_name="s")

    @functools.partial(
        pl.kernel,
        mesh=mesh,
        out_type=jax.ShapeDtypeStruct((E, 3 * D), jnp.float32),
        scratch_types=[
            pltpu.VMEM((e_per_w,), jnp.int32),
            pltpu.VMEM((e_per_w,), jnp.int32),
        ] + [pltpu.VMEM((CHUNK, D), jnp.float32)] * 6 + [
            pltpu.SemaphoreType.DMA,
            pltpu.SemaphoreType.DMA,
            pltpu.SemaphoreType.DMA,
            pltpu.SemaphoreType.DMA,
        ],
    )
    def sc_gather(edge_hbm, node_hbm, src_hbm, dst_hbm, out_hbm,
                  srcv, dstv, ev0, sv0, dv0, ev1, sv1, dv1,
                  ls0, ls1, ws0, ws1):
        wid = lax.axis_index("s") * 2 + lax.axis_index("c")
        base = wid * e_per_w
        pltpu.sync_copy(src_hbm.at[pl.ds(base, e_per_w)], srcv)
        pltpu.sync_copy(dst_hbm.at[pl.ds(base, e_per_w)], dstv)

        bufs = ((ev0, sv0, dv0, ls0, ws0), (ev1, sv1, dv1, ls1, ws1))

        def load_copies(c, b):
            ev, sv, dv, ls, _ = bufs[b]
            off = c * CHUNK
            row = base + off
            return (
                pltpu.make_async_copy(
                    edge_hbm.at[pl.ds(row, CHUNK)], ev, ls),
                pltpu.make_async_copy(
                    node_hbm.at[srcv.at[pl.ds(off, CHUNK)]], sv, ls),
                pltpu.make_async_copy(
                    node_hbm.at[dstv.at[pl.ds(off, CHUNK)]], dv, ls),
            )

        def write_copies(c, b):
            ev, sv, dv, _, ws = bufs[b]
            row = base + c * CHUNK
            return (
                pltpu.make_async_copy(
                    ev, out_hbm.at[pl.ds(row, CHUNK), pl.ds(0, D)], ws),
                pltpu.make_async_copy(
                    sv, out_hbm.at[pl.ds(row, CHUNK), pl.ds(D, D)], ws),
                pltpu.make_async_copy(
                    dv, out_hbm.at[pl.ds(row, CHUNK), pl.ds(2 * D, D)], ws),
            )

        def start(copies):
            for cp in copies:
                cp.start()

        def wait(copies):
            for cp in copies:
                cp.wait()

        start(load_copies(0, 0))
        start(load_copies(1, 1))

        def pair(p, carry):
            c0 = 2 * p
            c1 = c0 + 1
            wait(load_copies(c0, 0))
            start(write_copies(c0, 0))

            @pl.when(c1 < n_chunks)
            def _():
                wait(load_copies(c1, 1))
                start(write_copies(c1, 1))

            wait(write_copies(c0, 0))

            @pl.when(c0 + 2 < n_chunks)
            def _():
                start(load_copies(c0 + 2, 0))

            @pl.when(c1 < n_chunks)
            def _():
                wait(write_copies(c1, 1))

            @pl.when(c1 + 2 < n_chunks)
            def _():
                start(load_copies(c1 + 2, 1))

            return carry

        lax.fori_loop(0, (n_chunks + 1) // 2, pair, 0)

    return sc_gather


def kernel(edge_feat, node_feat, edge_index):
    E, D = edge_feat.shape
    N = node_feat.shape[0]
    src = edge_index[0].astype(jnp.int32)
    dst = edge_index[1].astype(jnp.int32)
    fn = _make_sc_kernel(E, N, D, NW=32, CHUNK=80)
    return fn(edge_feat, node_feat, src, dst)
